# Initial kernel scaffold; baseline (speedup 1.0000x reference)
import jax
import jax.numpy as jnp
from jax.experimental import pallas as pl


def kernel(x, edge_index, W, b):
    raise NotImplementedError("write your pallas kernel here")


# speedup vs baseline: 17.8086x; 17.8086x over previous
import functools

import jax
import jax.numpy as jnp
from jax import lax
from jax.experimental import pallas as pl
from jax.experimental.pallas import tpu as pltpu
from jax.experimental.pallas import tpu_sc as plsc

N = 10000
E = 320000
D = 128

NC = 2
NS = 16
CHUNK = 128
NCHUNKS = E // CHUNK
ROWS_PER_TILE = N // NS
TC_BLOCK = 1000

_mesh = plsc.VectorSubcoreMesh(core_axis_name="c", subcore_axis_name="s")


@functools.partial(
    pl.kernel,
    out_type=jax.ShapeDtypeStruct((2 * N,), jnp.float32),
    mesh=_mesh,
    scratch_types=[
        pltpu.VMEM((CHUNK,), jnp.int32),
        pltpu.VMEM((CHUNK,), jnp.float32),
        pltpu.VMEM((2000,), jnp.float32),
        pltpu.VMEM_SHARED((N,), jnp.float32),
    ],
)
def _deg_kernel(row_hbm, deg_hbm, idx_v, ones_v, zeros_v, acc_sh):
    cid = lax.axis_index("c")
    sid = lax.axis_index("s")

    one16 = jnp.ones((16,), jnp.float32)
    zero16 = jnp.zeros((16,), jnp.float32)
    for j in range(CHUNK // 16):
        ones_v[pl.ds(j * 16, 16)] = one16

    def fill_zero(i, _):
        zeros_v[pl.ds(i * 16, 16)] = zero16
        return 0

    lax.fori_loop(0, 2000 // 16, fill_zero, 0)

    @pl.when(sid < 5)
    def _():
        pltpu.sync_copy(zeros_v, acc_sh.at[pl.ds(sid * 2000, 2000)])

    plsc.subcore_barrier()

    n_iter = (1250 - sid + 15) // 16

    def body(k, _):
        chunk = cid * 1250 + sid + 16 * k
        pltpu.sync_copy(row_hbm.at[pl.ds(chunk * CHUNK, CHUNK)], idx_v)
        pltpu.sync_copy(ones_v, acc_sh.at[idx_v], add=True)
        return 0

    lax.fori_loop(0, n_iter, body, 0)

    plsc.subcore_barrier()

    @pl.when(sid < 10)
    def _():
        pltpu.sync_copy(acc_sh.at[pl.ds(sid * 1000, 1000)], zeros_v.at[pl.ds(0, 1000)])
        pltpu.sync_copy(
            zeros_v.at[pl.ds(0, 1000)],
            deg_hbm.at[pl.ds(cid * N + sid * 1000, 1000)],
        )


def _linear_body(x_ref, w_ref, b_ref, d0_ref, d1_ref, h_ref, r_ref):
    deg = d0_ref[:, 0] + d1_ref[:, 0]
    r_full = lax.rsqrt(deg)
    r_safe = jnp.where(deg > 0.0, r_full, 0.0)
    m = lax.dot_general(
        x_ref[...], w_ref[...],
        dimension_numbers=(((1,), (1,)), ((), ())),
        preferred_element_type=jnp.float32,
    )
    h_ref[...] = r_full[:, None] * (m + b_ref[0, :][None, :])
    r_ref[...] = r_safe[:, None]


def _linear(x, W, b2, d0, d1):
    grid = (N // TC_BLOCK,)
    return pl.pallas_call(
        _linear_body,
        grid=grid,
        in_specs=[
            pl.BlockSpec((TC_BLOCK, D), lambda i: (i, 0)),
            pl.BlockSpec((D, D), lambda i: (0, 0)),
            pl.BlockSpec((1, D), lambda i: (0, 0)),
            pl.BlockSpec((TC_BLOCK, 1), lambda i: (i, 0)),
            pl.BlockSpec((TC_BLOCK, 1), lambda i: (i, 0)),
        ],
        out_specs=[
            pl.BlockSpec((TC_BLOCK, D), lambda i: (i, 0)),
            pl.BlockSpec((TC_BLOCK, 1), lambda i: (i, 0)),
        ],
        out_shape=[
            jax.ShapeDtypeStruct((N, D), jnp.float32),
            jax.ShapeDtypeStruct((N, 1), jnp.float32),
        ],
    )(x, W, b2, d0, d1)


@functools.partial(
    pl.kernel,
    out_type=jax.ShapeDtypeStruct((2 * N, D), jnp.float32),
    mesh=_mesh,
    scratch_types=[
        pltpu.VMEM((CHUNK,), jnp.int32),
        pltpu.VMEM((CHUNK,), jnp.int32),
        pltpu.VMEM((CHUNK, D), jnp.float32),
        pltpu.VMEM((200, D), jnp.float32),
        pltpu.VMEM_SHARED((N, D), jnp.float32),
        pltpu.SemaphoreType.DMA,
    ],
)
def _agg_kernel(h_hbm, row_hbm, col_hbm, part_hbm,
                dst_v, src_v, rows_v, wb_v, acc_sh, sem):
    cid = lax.axis_index("c")
    sid = lax.axis_index("s")
    wid = sid * NC + cid

    zero16 = jnp.zeros((16,), jnp.float32)

    def zrow(i, _):
        for j in range(D // 16):
            wb_v[i, pl.ds(j * 16, 16)] = zero16
        return 0

    lax.fori_loop(0, 200, zrow, 0)

    n_z = (50 - sid + 15) // 16

    def zchunk(k, _):
        c = sid + 16 * k
        pltpu.sync_copy(wb_v, acc_sh.at[pl.ds(c * 200, 200)])
        return 0

    lax.fori_loop(0, n_z, zchunk, 0)

    plsc.subcore_barrier()

    n_iter = (NCHUNKS - wid + 31) // 32

    def body(k, _):
        chunk = wid + 32 * k
        base = chunk * CHUNK
        pltpu.sync_copy(row_hbm.at[pl.ds(base, CHUNK)], dst_v)
        pltpu.sync_copy(col_hbm.at[pl.ds(base, CHUNK)], src_v)
        pltpu.async_copy(h_hbm.at[src_v], rows_v, sem).wait()
        pltpu.sync_copy(rows_v, acc_sh.at[dst_v], add=True)
        return 0

    lax.fori_loop(0, n_iter, body, 0)

    plsc.subcore_barrier()

    def wchunk(k, _):
        c = sid + 16 * k
        pltpu.sync_copy(acc_sh.at[pl.ds(c * 200, 200)], wb_v)
        pltpu.sync_copy(wb_v, part_hbm.at[pl.ds(cid * N + c * 200, 200)])
        return 0

    lax.fori_loop(0, n_z, wchunk, 0)


def _combine_body(p0_ref, p1_ref, r_ref, o_ref):
    o_ref[...] = (p0_ref[...] + p1_ref[...]) * r_ref[...]


def _combine(p0, p1, r):
    grid = (N // TC_BLOCK,)
    return pl.pallas_call(
        _combine_body,
        grid=grid,
        in_specs=[
            pl.BlockSpec((TC_BLOCK, D), lambda i: (i, 0)),
            pl.BlockSpec((TC_BLOCK, D), lambda i: (i, 0)),
            pl.BlockSpec((TC_BLOCK, 1), lambda i: (i, 0)),
        ],
        out_specs=pl.BlockSpec((TC_BLOCK, D), lambda i: (i, 0)),
        out_shape=jax.ShapeDtypeStruct((N, D), jnp.float32),
    )(p0, p1, r)


@jax.jit
def _impl(x, row, col, W, b):
    degp = _deg_kernel(row)
    d0 = degp[:N].reshape(N, 1)
    d1 = degp[N:].reshape(N, 1)
    h, r = _linear(x, W, b.reshape(1, D), d0, d1)
    parts = _agg_kernel(h, row, col)
    return _combine(parts[:N], parts[N:], r)


def kernel(x, edge_index, W, b):
    row = jnp.asarray(edge_index[0], jnp.int32)
    col = jnp.asarray(edge_index[1], jnp.int32)
    return _impl(x, row, col, W, b)


# speedup vs baseline: 29.1242x; 1.6354x over previous
import functools

import jax
import jax.numpy as jnp
from jax import lax
from jax.experimental import pallas as pl
from jax.experimental.pallas import tpu as pltpu
from jax.experimental.pallas import tpu_sc as plsc

N = 10000
E = 320000
D = 128

NC = 2
NS = 16
NW = NC * NS
CHUNK = 128
NCHUNKS = E // CHUNK
GSZ = 8
BLK = GSZ * CHUNK
NGROUPS = NCHUNKS // GSZ
NTAIL = NCHUNKS - NGROUPS * GSZ
TC_BLOCK = 1000

_mesh = plsc.VectorSubcoreMesh(core_axis_name="c", subcore_axis_name="s")


def _copy128(blk_ref, off, dst_ref):
    for i in range(CHUNK // 16):
        dst_ref[pl.ds(i * 16, 16)] = blk_ref[pl.ds(off + i * 16, 16)]


@functools.partial(
    pl.kernel,
    out_type=jax.ShapeDtypeStruct((2 * N,), jnp.float32),
    mesh=_mesh,
    scratch_types=[
        pltpu.VMEM((BLK,), jnp.int32),
        pltpu.VMEM((CHUNK,), jnp.int32),
        pltpu.VMEM((CHUNK,), jnp.float32),
        pltpu.VMEM((2000,), jnp.float32),
        pltpu.VMEM_SHARED((N,), jnp.float32),
    ],
)
def _deg_kernel(row_hbm, deg_hbm, blk_v, idx_v, ones_v, zeros_v, acc_sh):
    cid = lax.axis_index("c")
    sid = lax.axis_index("s")
    wid = sid * NC + cid

    one16 = jnp.ones((16,), jnp.float32)
    zero16 = jnp.zeros((16,), jnp.float32)
    for j in range(CHUNK // 16):
        ones_v[pl.ds(j * 16, 16)] = one16

    def fill_zero(i, _):
        zeros_v[pl.ds(i * 16, 16)] = zero16
        return 0

    lax.fori_loop(0, 2000 // 16, fill_zero, 0)

    @pl.when(sid < 5)
    def _():
        pltpu.sync_copy(zeros_v, acc_sh.at[pl.ds(sid * 2000, 2000)])

    plsc.subcore_barrier()

    n_g = (NGROUPS - wid + NW - 1) // NW

    def body(t, _):
        g = wid + NW * t
        pltpu.sync_copy(row_hbm.at[pl.ds(g * BLK, BLK)], blk_v)
        for j in range(GSZ):
            _copy128(blk_v, j * CHUNK, idx_v)
            pltpu.sync_copy(ones_v, acc_sh.at[idx_v], add=True)
        return 0

    lax.fori_loop(0, n_g, body, 0)

    @pl.when(wid < NTAIL)
    def _():
        base = NGROUPS * BLK + wid * CHUNK
        pltpu.sync_copy(row_hbm.at[pl.ds(base, CHUNK)], idx_v)
        pltpu.sync_copy(ones_v, acc_sh.at[idx_v], add=True)

    plsc.subcore_barrier()

    @pl.when(sid < 10)
    def _():
        pltpu.sync_copy(acc_sh.at[pl.ds(sid * 1000, 1000)],
                        zeros_v.at[pl.ds(0, 1000)])
        pltpu.sync_copy(zeros_v.at[pl.ds(0, 1000)],
                        deg_hbm.at[pl.ds(cid * N + sid * 1000, 1000)])


def _linear_body(x_ref, w_ref, b_ref, d0_ref, d1_ref, h_ref, r_ref):
    deg = d0_ref[:, 0] + d1_ref[:, 0]
    r_full = lax.rsqrt(deg)
    r_safe = jnp.where(deg > 0.0, r_full, 0.0)
    m = lax.dot_general(
        x_ref[...], w_ref[...],
        dimension_numbers=(((1,), (1,)), ((), ())),
        preferred_element_type=jnp.float32,
    )
    h_ref[...] = r_full[:, None] * (m + b_ref[0, :][None, :])
    r_ref[...] = r_safe[:, None]


def _linear(x, W, b2, d0, d1):
    grid = (N // TC_BLOCK,)
    return pl.pallas_call(
        _linear_body,
        grid=grid,
        in_specs=[
            pl.BlockSpec((TC_BLOCK, D), lambda i: (i, 0)),
            pl.BlockSpec((D, D), lambda i: (0, 0)),
            pl.BlockSpec((1, D), lambda i: (0, 0)),
            pl.BlockSpec((TC_BLOCK, 1), lambda i: (i, 0)),
            pl.BlockSpec((TC_BLOCK, 1), lambda i: (i, 0)),
        ],
        out_specs=[
            pl.BlockSpec((TC_BLOCK, D), lambda i: (i, 0)),
            pl.BlockSpec((TC_BLOCK, 1), lambda i: (i, 0)),
        ],
        out_shape=[
            jax.ShapeDtypeStruct((N, D), jnp.float32),
            jax.ShapeDtypeStruct((N, 1), jnp.float32),
        ],
    )(x, W, b2, d0, d1)


@functools.partial(
    pl.kernel,
    out_type=jax.ShapeDtypeStruct((2 * N, D), jnp.float32),
    mesh=_mesh,
    scratch_types=[
        pltpu.VMEM((BLK,), jnp.int32),
        pltpu.VMEM((BLK,), jnp.int32),
        pltpu.VMEM((CHUNK,), jnp.int32),
        pltpu.VMEM((CHUNK,), jnp.int32),
        pltpu.VMEM((CHUNK,), jnp.int32),
        pltpu.VMEM((CHUNK,), jnp.int32),
        pltpu.VMEM((CHUNK, D), jnp.float32),
        pltpu.VMEM((CHUNK, D), jnp.float32),
        pltpu.VMEM((40, D), jnp.float32),
        pltpu.VMEM_SHARED((N, D), jnp.float32),
        pltpu.SemaphoreType.DMA,
        pltpu.SemaphoreType.DMA,
    ],
)
def _agg_kernel(h_hbm, row_hbm, col_hbm, part_hbm,
                dst_blk, src_blk, dst0, dst1, src0, src1, rows0, rows1,
                wb_v, acc_sh, gsem0, gsem1):
    cid = lax.axis_index("c")
    sid = lax.axis_index("s")
    wid = sid * NC + cid

    rows = (rows0, rows1)
    dsts = (dst0, dst1)
    srcs = (src0, src1)
    gsems = (gsem0, gsem1)

    zero16 = jnp.zeros((16,), jnp.float32)

    def zrow(i, _):
        for j in range(D // 16):
            wb_v[i, pl.ds(j * 16, 16)] = zero16
        return 0

    lax.fori_loop(0, 40, zrow, 0)

    n_z = (250 - sid + 15) // 16

    def zchunk(k, _):
        c = sid + 16 * k
        pltpu.sync_copy(wb_v, acc_sh.at[pl.ds(c * 40, 40)])
        return 0

    lax.fori_loop(0, n_z, zchunk, 0)

    plsc.subcore_barrier()

    n_g = (NGROUPS - wid + NW - 1) // NW

    def body(t, _):
        g = wid + NW * t
        pltpu.sync_copy(row_hbm.at[pl.ds(g * BLK, BLK)], dst_blk)
        pltpu.sync_copy(col_hbm.at[pl.ds(g * BLK, BLK)], src_blk)

        gd = [None] * GSZ
        _copy128(src_blk, 0, src0)
        gd[0] = pltpu.async_copy(h_hbm.at[src0], rows[0], gsems[0])
        for j in range(GSZ):
            b = j % 2
            if j + 1 < GSZ:
                _copy128(src_blk, (j + 1) * CHUNK, srcs[1 - b])
                gd[j + 1] = pltpu.async_copy(
                    h_hbm.at[srcs[1 - b]], rows[1 - b], gsems[1 - b])
            _copy128(dst_blk, j * CHUNK, dsts[b])
            gd[j].wait()
            pltpu.sync_copy(rows[b], acc_sh.at[dsts[b]], add=True)
        return 0

    lax.fori_loop(0, n_g, body, 0)

    @pl.when(wid < NTAIL)
    def _():
        base = NGROUPS * BLK + wid * CHUNK
        pltpu.sync_copy(row_hbm.at[pl.ds(base, CHUNK)], dst0)
        pltpu.sync_copy(col_hbm.at[pl.ds(base, CHUNK)], src0)
        pltpu.async_copy(h_hbm.at[src0], rows0, gsem0).wait()
        pltpu.sync_copy(rows0, acc_sh.at[dst0], add=True)

    plsc.subcore_barrier()

    def wchunk(k, _):
        c = sid + 16 * k
        pltpu.sync_copy(acc_sh.at[pl.ds(c * 40, 40)], wb_v)
        pltpu.sync_copy(wb_v, part_hbm.at[pl.ds(cid * N + c * 40, 40)])
        return 0

    lax.fori_loop(0, n_z, wchunk, 0)


def _combine_body(p0_ref, p1_ref, r_ref, o_ref):
    o_ref[...] = (p0_ref[...] + p1_ref[...]) * r_ref[...]


def _combine(p0, p1, r):
    grid = (N // TC_BLOCK,)
    return pl.pallas_call(
        _combine_body,
        grid=grid,
        in_specs=[
            pl.BlockSpec((TC_BLOCK, D), lambda i: (i, 0)),
            pl.BlockSpec((TC_BLOCK, D), lambda i: (i, 0)),
            pl.BlockSpec((TC_BLOCK, 1), lambda i: (i, 0)),
        ],
        out_specs=pl.BlockSpec((TC_BLOCK, D), lambda i: (i, 0)),
        out_shape=jax.ShapeDtypeStruct((N, D), jnp.float32),
    )(p0, p1, r)


@jax.jit
def _impl(x, row, col, W, b):
    degp = _deg_kernel(row)
    d0 = degp[:N].reshape(N, 1)
    d1 = degp[N:].reshape(N, 1)
    h, r = _linear(x, W, b.reshape(1, D), d0, d1)
    parts = _agg_kernel(h, row, col)
    return _combine(parts[:N], parts[N:], r)


def kernel(x, edge_index, W, b):
    row = jnp.asarray(edge_index[0], jnp.int32)
    col = jnp.asarray(edge_index[1], jnp.int32)
    return _impl(x, row, col, W, b)


# speedup vs baseline: 29.7360x; 1.0210x over previous
import functools

import jax
import jax.numpy as jnp
from jax import lax
from jax.experimental import pallas as pl
from jax.experimental.pallas import tpu as pltpu
from jax.experimental.pallas import tpu_sc as plsc

N = 10000
E = 320000
D = 128

NC = 2
NS = 16
NW = NC * NS
CHUNK = 128
NCHUNKS = E // CHUNK
GSZ = 32
BLK = GSZ * CHUNK
NGROUPS = NCHUNKS // GSZ
NTAIL = NCHUNKS - NGROUPS * GSZ
TC_BLOCK = 1000

_mesh = plsc.VectorSubcoreMesh(core_axis_name="c", subcore_axis_name="s")


def _copy128(blk_ref, off, dst_ref):
    for i in range(CHUNK // 16):
        dst_ref[pl.ds(i * 16, 16)] = blk_ref[pl.ds(off + i * 16, 16)]


@functools.partial(
    pl.kernel,
    out_type=jax.ShapeDtypeStruct((2 * N,), jnp.float32),
    mesh=_mesh,
    scratch_types=[
        pltpu.VMEM((BLK,), jnp.int32),
        pltpu.VMEM((CHUNK,), jnp.int32),
        pltpu.VMEM((CHUNK,), jnp.float32),
        pltpu.VMEM((2000,), jnp.float32),
        pltpu.VMEM_SHARED((N,), jnp.float32),
    ],
)
def _deg_kernel(row_hbm, deg_hbm, blk_v, idx_v, ones_v, zeros_v, acc_sh):
    cid = lax.axis_index("c")
    sid = lax.axis_index("s")
    wid = sid * NC + cid

    one16 = jnp.ones((16,), jnp.float32)
    zero16 = jnp.zeros((16,), jnp.float32)
    for j in range(CHUNK // 16):
        ones_v[pl.ds(j * 16, 16)] = one16

    def fill_zero(i, _):
        zeros_v[pl.ds(i * 16, 16)] = zero16
        return 0

    lax.fori_loop(0, 2000 // 16, fill_zero, 0)

    @pl.when(sid < 5)
    def _():
        pltpu.sync_copy(zeros_v, acc_sh.at[pl.ds(sid * 2000, 2000)])

    plsc.subcore_barrier()

    n_g = (NGROUPS - wid + NW - 1) // NW

    def body(t, _):
        g = wid + NW * t
        pltpu.sync_copy(row_hbm.at[pl.ds(g * BLK, BLK)], blk_v)
        for j in range(GSZ):
            _copy128(blk_v, j * CHUNK, idx_v)
            pltpu.sync_copy(ones_v, acc_sh.at[idx_v], add=True)
        return 0

    lax.fori_loop(0, n_g, body, 0)

    @pl.when(wid < NTAIL)
    def _():
        base = NGROUPS * BLK + wid * CHUNK
        pltpu.sync_copy(row_hbm.at[pl.ds(base, CHUNK)], idx_v)
        pltpu.sync_copy(ones_v, acc_sh.at[idx_v], add=True)

    plsc.subcore_barrier()

    @pl.when(sid < 10)
    def _():
        pltpu.sync_copy(acc_sh.at[pl.ds(sid * 1000, 1000)],
                        zeros_v.at[pl.ds(0, 1000)])
        pltpu.sync_copy(zeros_v.at[pl.ds(0, 1000)],
                        deg_hbm.at[pl.ds(cid * N + sid * 1000, 1000)])


def _linear_body(x_ref, w_ref, b_ref, d0_ref, d1_ref, h_ref, r_ref):
    deg = d0_ref[:, 0] + d1_ref[:, 0]
    r_full = lax.rsqrt(deg)
    r_safe = jnp.where(deg > 0.0, r_full, 0.0)
    m = lax.dot_general(
        x_ref[...], w_ref[...],
        dimension_numbers=(((1,), (1,)), ((), ())),
        preferred_element_type=jnp.float32,
    )
    h_ref[...] = r_full[:, None] * (m + b_ref[0, :][None, :])
    r_ref[...] = r_safe[:, None]


def _linear(x, W, b2, d0, d1):
    grid = (N // TC_BLOCK,)
    return pl.pallas_call(
        _linear_body,
        grid=grid,
        in_specs=[
            pl.BlockSpec((TC_BLOCK, D), lambda i: (i, 0)),
            pl.BlockSpec((D, D), lambda i: (0, 0)),
            pl.BlockSpec((1, D), lambda i: (0, 0)),
            pl.BlockSpec((TC_BLOCK, 1), lambda i: (i, 0)),
            pl.BlockSpec((TC_BLOCK, 1), lambda i: (i, 0)),
        ],
        out_specs=[
            pl.BlockSpec((TC_BLOCK, D), lambda i: (i, 0)),
            pl.BlockSpec((TC_BLOCK, 1), lambda i: (i, 0)),
        ],
        out_shape=[
            jax.ShapeDtypeStruct((N, D), jnp.float32),
            jax.ShapeDtypeStruct((N, 1), jnp.float32),
        ],
    )(x, W, b2, d0, d1)


@functools.partial(
    pl.kernel,
    out_type=jax.ShapeDtypeStruct((2 * N, D), jnp.float32),
    mesh=_mesh,
    scratch_types=[
        pltpu.VMEM((BLK,), jnp.int32),
        pltpu.VMEM((BLK,), jnp.int32),
        pltpu.VMEM((CHUNK,), jnp.int32),
        pltpu.VMEM((CHUNK,), jnp.int32),
        pltpu.VMEM((CHUNK,), jnp.int32),
        pltpu.VMEM((CHUNK,), jnp.int32),
        pltpu.VMEM((CHUNK, D), jnp.float32),
        pltpu.VMEM((CHUNK, D), jnp.float32),
        pltpu.VMEM((40, D), jnp.float32),
        pltpu.VMEM_SHARED((N, D), jnp.float32),
        pltpu.SemaphoreType.DMA,
        pltpu.SemaphoreType.DMA,
    ],
)
def _agg_kernel(h_hbm, row_hbm, col_hbm, part_hbm,
                dst_blk, src_blk, dst0, dst1, src0, src1, rows0, rows1,
                wb_v, acc_sh, gsem0, gsem1):
    cid = lax.axis_index("c")
    sid = lax.axis_index("s")
    wid = sid * NC + cid

    rows = (rows0, rows1)
    dsts = (dst0, dst1)
    srcs = (src0, src1)
    gsems = (gsem0, gsem1)

    zero16 = jnp.zeros((16,), jnp.float32)

    def zrow(i, _):
        for j in range(D // 16):
            wb_v[i, pl.ds(j * 16, 16)] = zero16
        return 0

    lax.fori_loop(0, 40, zrow, 0)

    n_z = (250 - sid + 15) // 16

    def zchunk(k, _):
        c = sid + 16 * k
        pltpu.sync_copy(wb_v, acc_sh.at[pl.ds(c * 40, 40)])
        return 0

    lax.fori_loop(0, n_z, zchunk, 0)

    plsc.subcore_barrier()

    n_g = (NGROUPS - wid + NW - 1) // NW

    def body(t, _):
        g = wid + NW * t
        d1 = pltpu.async_copy(row_hbm.at[pl.ds(g * BLK, BLK)], dst_blk, gsem0)
        d2 = pltpu.async_copy(col_hbm.at[pl.ds(g * BLK, BLK)], src_blk, gsem1)
        d1.wait()
        d2.wait()

        gd = [None] * GSZ
        _copy128(src_blk, 0, src0)
        gd[0] = pltpu.async_copy(h_hbm.at[src0], rows[0], gsems[0])
        for j in range(GSZ):
            b = j % 2
            if j + 1 < GSZ:
                _copy128(src_blk, (j + 1) * CHUNK, srcs[1 - b])
                gd[j + 1] = pltpu.async_copy(
                    h_hbm.at[srcs[1 - b]], rows[1 - b], gsems[1 - b])
            _copy128(dst_blk, j * CHUNK, dsts[b])
            gd[j].wait()
            pltpu.sync_copy(rows[b], acc_sh.at[dsts[b]], add=True)
        return 0

    lax.fori_loop(0, n_g, body, 0)

    @pl.when(wid < NTAIL)
    def _():
        base = NGROUPS * BLK + wid * CHUNK
        pltpu.sync_copy(row_hbm.at[pl.ds(base, CHUNK)], dst0)
        pltpu.sync_copy(col_hbm.at[pl.ds(base, CHUNK)], src0)
        pltpu.async_copy(h_hbm.at[src0], rows0, gsem0).wait()
        pltpu.sync_copy(rows0, acc_sh.at[dst0], add=True)

    plsc.subcore_barrier()

    def wchunk(k, _):
        c = sid + 16 * k
        pltpu.sync_copy(acc_sh.at[pl.ds(c * 40, 40)], wb_v)
        pltpu.sync_copy(wb_v, part_hbm.at[pl.ds(cid * N + c * 40, 40)])
        return 0

    lax.fori_loop(0, n_z, wchunk, 0)


def _combine_body(p0_ref, p1_ref, r_ref, o_ref):
    o_ref[...] = (p0_ref[...] + p1_ref[...]) * r_ref[...]


def _combine(p0, p1, r):
    grid = (N // TC_BLOCK,)
    return pl.pallas_call(
        _combine_body,
        grid=grid,
        in_specs=[
            pl.BlockSpec((TC_BLOCK, D), lambda i: (i, 0)),
            pl.BlockSpec((TC_BLOCK, D), lambda i: (i, 0)),
            pl.BlockSpec((TC_BLOCK, 1), lambda i: (i, 0)),
        ],
        out_specs=pl.BlockSpec((TC_BLOCK, D), lambda i: (i, 0)),
        out_shape=jax.ShapeDtypeStruct((N, D), jnp.float32),
    )(p0, p1, r)


@jax.jit
def _impl(x, row, col, W, b):
    degp = _deg_kernel(row)
    d0 = degp[:N].reshape(N, 1)
    d1 = degp[N:].reshape(N, 1)
    h, r = _linear(x, W, b.reshape(1, D), d0, d1)
    parts = _agg_kernel(h, row, col)
    return _combine(parts[:N], parts[N:], r)


def kernel(x, edge_index, W, b):
    row = jnp.asarray(edge_index[0], jnp.int32)
    col = jnp.asarray(edge_index[1], jnp.int32)
    return _impl(x, row, col, W, b)
